# Initial kernel scaffold; baseline (speedup 1.0000x reference)
import jax
import jax.numpy as jnp
from jax.experimental import pallas as pl


def kernel(date, table):
    raise NotImplementedError("write your pallas kernel here")


# speedup vs baseline: 9.2938x; 9.2938x over previous
import jax
import jax.numpy as jnp
from jax.experimental import pallas as pl


def _embed_kernel(date_ref, table_ref, out_ref):
    d = date_ref[...][..., None]
    t = table_ref[...]
    out_ref[...] = jnp.where(
        d < 2,
        jnp.where(d == 0, t[0], t[1]),
        jnp.where(d == 2, t[2], t[3]),
    )


def kernel(date, table):
    n, c = date.shape
    e = table.shape[1]
    block_rows = 128
    grid = (n // block_rows,)
    return pl.pallas_call(
        _embed_kernel,
        grid=grid,
        in_specs=[
            pl.BlockSpec((block_rows, c), lambda i: (i, 0)),
            pl.BlockSpec((4, e), lambda i: (0, 0)),
        ],
        out_specs=pl.BlockSpec((block_rows, c, e), lambda i: (i, 0, 0)),
        out_shape=jax.ShapeDtypeStruct((n, c, e), table.dtype),
    )(date, table)


# speedup vs baseline: 11.6538x; 1.2539x over previous
import jax
import jax.numpy as jnp
from jax.experimental import pallas as pl


def _embed_kernel(date_ref, table_ref, out_ref):
    d = date_ref[...]
    r, c = d.shape
    e = table_ref.shape[1] // c
    drep = jnp.repeat(d, e, axis=1)
    t = table_ref[...]
    out_ref[...] = jnp.where(
        drep < 2,
        jnp.where(drep == 0, t[0], t[1]),
        jnp.where(drep == 2, t[2], t[3]),
    )


def kernel(date, table):
    n, c = date.shape
    e = table.shape[1]
    table_tiled = jnp.tile(table, (1, c))
    block_rows = 256
    grid = (n // block_rows,)
    out2 = pl.pallas_call(
        _embed_kernel,
        grid=grid,
        in_specs=[
            pl.BlockSpec((block_rows, c), lambda i: (i, 0)),
            pl.BlockSpec((4, c * e), lambda i: (0, 0)),
        ],
        out_specs=pl.BlockSpec((block_rows, c * e), lambda i: (i, 0)),
        out_shape=jax.ShapeDtypeStruct((n, c * e), table.dtype),
    )(date, table_tiled)
    return out2.reshape(n, c, e)
